# baseline (speedup 1.0000x reference)
import functools

import jax
import jax.numpy as jnp
from jax import lax
from jax.experimental import pallas as pl
from jax.experimental.pallas import tpu as pltpu
from jax.experimental.pallas import tpu_sc as plsc

_B = 4096
_D = 32
_NU = 1000000
_NI = 100000
_BM = 512
_NBUF = 8


@functools.lru_cache(maxsize=None)
def _make_gather():
    info = plsc.get_sparse_core_info()
    nc, ns = info.num_cores, info.num_subcores
    nw = nc * ns
    bpw = _B // nw

    mesh = plsc.VectorSubcoreMesh(core_axis_name="c", subcore_axis_name="s")

    @functools.partial(
        pl.kernel,
        mesh=mesh,
        out_type=(
            jax.ShapeDtypeStruct((_B, _D), jnp.float32),
            jax.ShapeDtypeStruct((_B, _D), jnp.float32),
        ),
        scratch_types=[
            pltpu.VMEM((bpw + 16,), jnp.int32),
            pltpu.VMEM((bpw, _D), jnp.float32),
        ]
        + [pltpu.VMEM((_D, 128), jnp.float32) for _ in range(_NBUF)]
        + [pltpu.SemaphoreType.DMA for _ in range(_NBUF)],
        compiler_params=pltpu.CompilerParams(
            use_tc_tiling_on_sc=True, needs_layout_passes=False),
    )
    def gather(user_ids, item_ids, ut, it, u_out, i_out, idx_v, rows_v, *bs):
        bufs, sems = bs[:_NBUF], bs[_NBUF:]
        wid = lax.axis_index("s") * nc + lax.axis_index("c")
        base = wid * bpw

        def phase(tbl, n_rows, ids_hbm, out_hbm):
            pltpu.sync_copy(ids_hbm.at[pl.ds(base, bpw)],
                            idx_v.at[pl.ds(0, bpw)])

            def uid_at(u):
                return idx_v[pl.ds(u, 16)][0]

            def start(u, k):
                j = pl.multiple_of(uid_at(u) & ~127, 128)
                pltpu.async_copy(tbl.at[:, pl.ds(j, 128)], bufs[k], sems[k])

            def extract(u, k):
                uid = uid_at(u)
                cv = jnp.full((16,), uid & 127, jnp.int32)
                r0 = lax.iota(jnp.int32, 16)
                pltpu.make_async_copy(tbl.at[:, pl.ds(0, 128)], bufs[k],
                                      sems[k]).wait()
                v0 = plsc.load_gather(bufs[k], [r0, cv])
                v1 = plsc.load_gather(bufs[k], [r0 + 16, cv])
                rows_v[u, pl.ds(0, 16)] = v0
                rows_v[u, pl.ds(16, 16)] = v1

            for k in range(_NBUF):
                start(k, k)

            def wave(w, carry):
                for k in range(_NBUF):
                    u = w * _NBUF + k
                    extract(u, k)
                    nu = u + _NBUF

                    @pl.when(nu < bpw)
                    def _():
                        start(nu, k)
                return carry

            lax.fori_loop(0, bpw // _NBUF, wave, 0, unroll=False)
            pltpu.sync_copy(rows_v, out_hbm.at[pl.ds(base, bpw)])

        phase(ut, _NU, user_ids, u_out)
        phase(it, _NI, item_ids, i_out)

    return gather


def _matmul_body(u_ref, i_ref, o_ref):
    o_ref[...] = lax.dot_general(
        u_ref[...], i_ref[...],
        (((1,), (1,)), ((), ())),
        preferred_element_type=jnp.float32,
    )


def _matmul(u, i):
    return pl.pallas_call(
        _matmul_body,
        grid=(_B // _BM,),
        in_specs=[
            pl.BlockSpec((_BM, _D), lambda m: (m, 0)),
            pl.BlockSpec((_B, _D), lambda m: (0, 0)),
        ],
        out_specs=pl.BlockSpec((_BM, _B), lambda m: (m, 0)),
        out_shape=jax.ShapeDtypeStruct((_B, _B), jnp.float32),
    )(u, i)


@jax.jit
def kernel(user_ids, item_ids, user_table, item_table):
    u, i = _make_gather()(user_ids, item_ids, user_table.T, item_table.T)
    return _matmul(u, i)
